# baseline (speedup 1.0000x reference)
import functools

import numpy as np
import jax
import jax.numpy as jnp
from jax import lax
from jax.experimental import pallas as pl
from jax.experimental.pallas import tpu as pltpu
from jax.experimental.pallas import tpu_sc as plsc

_N, _H, _W = 64, 512, 512
_S = 112 * 112 * 3
_P = 112 * 112
_K = int(0.75 * _P)
_R = _P - _K
_NB = 8
_BAND_ROWS = 72
_ROW0 = np.array([min(b * 64, _H - _BAND_ROWS) for b in range(_NB)], dtype=np.int32)


def _threefry2x32(k1, k2, x0, x1):
    k1 = np.uint32(k1)
    k2 = np.uint32(k2)
    x0 = x0.astype(np.uint32).copy()
    x1 = x1.astype(np.uint32).copy()
    ks = [k1, k2, np.uint32(k1 ^ k2 ^ np.uint32(0x1BD11BDA))]
    rot = [(13, 15, 26, 6), (17, 29, 16, 24)]
    x0 = x0 + ks[0]
    x1 = x1 + ks[1]
    for g in range(5):
        for r in rot[g % 2]:
            x0 = x0 + x1
            x1 = (x1 << np.uint32(r)) | (x1 >> np.uint32(32 - r))
            x1 = x0 ^ x1
        x0 = x0 + ks[(g + 1) % 3]
        x1 = x1 + ks[(g + 2) % 3] + np.uint32(g + 1)
    return x0, x1


def _np_uniform(rawkey, shape):
    size = int(np.prod(shape))
    b1, b2 = _threefry2x32(rawkey[0], rawkey[1],
                           np.zeros(size, np.uint32),
                           np.arange(size, dtype=np.uint32))
    bits = b1 ^ b2
    fb = (bits >> np.uint32(9)) | np.uint32(0x3F800000)
    return (fb.view(np.float32) - np.float32(1.0)).reshape(shape)


def _build_tables():
    b1, b2 = _threefry2x32(np.uint32(0), np.uint32(42),
                           np.zeros(2, np.uint32), np.arange(2, dtype=np.uint32))
    pc = _np_uniform((b1[0], b2[0]), (_N, _S, 2))
    rc = _np_uniform((b1[1], b2[1]), (_N, _R, 2))
    coords = np.concatenate([pc, rc], axis=1)

    x = coords[..., 0] * np.float32(_W) - np.float32(0.5)
    y = coords[..., 1] * np.float32(_H) - np.float32(0.5)
    x0 = np.floor(x)
    y0 = np.floor(y)
    fx1 = x - x0
    fx0 = np.float32(1.0) - fx1
    fy1 = y - y0
    fy0 = np.float32(1.0) - fy1

    def slot_weights(c0, f0, f1, lim):
        base = np.clip(c0, 0.0, lim - 2.0).astype(np.int32)
        g = np.zeros(c0.shape + (2,), dtype=np.float32)
        for d, f in ((0, f0), (1, f1)):
            ic = c0 + d
            valid = (ic >= 0) & (ic <= lim - 1)
            slot = np.clip(ic.astype(np.int64) - base, 0, 1).astype(np.int32)
            for s_ in (0, 1):
                g[..., s_] += np.where(valid & (slot == s_), f, np.float32(0.0))
        return base, g

    basex, gx = slot_weights(x0, fx0, fx1, _W)
    basey, gy = slot_weights(y0, fy0, fy1, _H)

    band = np.minimum(basey >> 6, _NB - 1).astype(np.int32)
    base_local = (basey - _ROW0[band]) * _W + basex
    ws = (gx[..., 0] * gy[..., 0], gx[..., 1] * gy[..., 0],
          gx[..., 0] * gy[..., 1], gx[..., 1] * gy[..., 1])

    counts = np.zeros((_N, _NB), np.int32)
    for n in range(_N):
        counts[n] = np.bincount(band[n], minlength=_NB)
    cap = int(counts.max())
    cap = (cap + 127) // 128 * 128

    tbl_base = np.zeros((_N, _NB, cap), np.int32)
    tbl_w = np.zeros((4, _N, _NB, cap), np.float32)
    code = np.zeros((_N, _NB, cap), np.float32)
    for n in range(_N):
        for b in range(_NB):
            i1 = np.nonzero(band[n, :_S] == b)[0]
            i2 = np.nonzero(band[n, _S:] == b)[0] + _S
            idx = np.concatenate([i1, i2])
            c = len(idx)
            tbl_base[n, b, :c] = base_local[n, idx]
            for k_ in range(4):
                tbl_w[k_, n, b, :c] = ws[k_][n, idx]
            code[n, b, :len(i1)] = 1.0
            code[n, b, len(i1):c] = 2.0
    return tbl_base, tbl_w, code, cap


_TBL_BASE, _TBL_W, _CODE, _C = _build_tables()
_M = _NB * _C
_CODEF = _CODE.reshape(_N * _M // 128, 128)

_TASKS_PER_TILE = (_N * _NB) // 32


_BAND_WORDS = _BAND_ROWS * _W


def _sc_sample_body(pred_hbm, target_hbm, base_hbm, w00_hbm, w01_hbm, w10_hbm,
                    w11_hbm, out_l_hbm, out_t_hbm,
                    band_p, band_t, base_v, w00_v, w01_v, w10_v, w11_v,
                    out_lv, out_tv):
    wid = lax.axis_index("s") * 2 + lax.axis_index("c")

    def task(t, carry):
        gid = wid * _TASKS_PER_TILE + t
        n = gid // _NB
        b = gid % _NB
        row0 = jnp.minimum(b * 64, _H - _BAND_ROWS)
        tbl_off = gid * _C
        pltpu.sync_copy(pred_hbm.at[n, pl.ds(row0, _BAND_ROWS), :], band_p)
        pltpu.sync_copy(target_hbm.at[n, pl.ds(row0, _BAND_ROWS), :], band_t)
        pltpu.sync_copy(base_hbm.at[pl.ds(tbl_off, _C)], base_v)
        pltpu.sync_copy(w00_hbm.at[pl.ds(tbl_off, _C)], w00_v)
        pltpu.sync_copy(w01_hbm.at[pl.ds(tbl_off, _C)], w01_v)
        pltpu.sync_copy(w10_hbm.at[pl.ds(tbl_off, _C)], w10_v)
        pltpu.sync_copy(w11_hbm.at[pl.ds(tbl_off, _C)], w11_v)

        def group(j, c2):
            sl = pl.ds(j * 16, 16)
            i00 = base_v[sl]
            by = lax.shift_right_logical(i00, 9)
            bx = lax.bitwise_and(i00, 511)
            by1 = by + 1
            bx1 = bx + 1
            a00 = w00_v[sl]
            a01 = w01_v[sl]
            a10 = w10_v[sl]
            a11 = w11_v[sl]
            out_lv[sl] = (plsc.load_gather(band_p, [by, bx]) * a00
                          + plsc.load_gather(band_p, [by, bx1]) * a01
                          + plsc.load_gather(band_p, [by1, bx]) * a10
                          + plsc.load_gather(band_p, [by1, bx1]) * a11)
            out_tv[sl] = (plsc.load_gather(band_t, [by, bx]) * a00
                          + plsc.load_gather(band_t, [by, bx1]) * a01
                          + plsc.load_gather(band_t, [by1, bx]) * a10
                          + plsc.load_gather(band_t, [by1, bx1]) * a11)
            return c2

        lax.fori_loop(0, _C // 16, group, 0)
        pltpu.sync_copy(out_lv, out_l_hbm.at[pl.ds(tbl_off, _C)])
        pltpu.sync_copy(out_tv, out_t_hbm.at[pl.ds(tbl_off, _C)])
        return carry

    lax.fori_loop(0, _TASKS_PER_TILE, task, 0)


@functools.cache
def _sc_sample():
    mesh = plsc.VectorSubcoreMesh(core_axis_name="c", subcore_axis_name="s",
                                  num_cores=2, num_subcores=16)
    return pl.kernel(
        _sc_sample_body,
        out_type=(jax.ShapeDtypeStruct((_N * _NB * _C,), jnp.float32),
                  jax.ShapeDtypeStruct((_N * _NB * _C,), jnp.float32)),
        mesh=mesh,
        compiler_params=pltpu.CompilerParams(needs_layout_passes=False),
        scratch_types=[
            pltpu.VMEM((_BAND_ROWS, _W), jnp.float32),
            pltpu.VMEM((_BAND_ROWS, _W), jnp.float32),
            pltpu.VMEM((_C,), jnp.int32),
            pltpu.VMEM((_C,), jnp.float32),
            pltpu.VMEM((_C,), jnp.float32),
            pltpu.VMEM((_C,), jnp.float32),
            pltpu.VMEM((_C,), jnp.float32),
            pltpu.VMEM((_C,), jnp.float32),
            pltpu.VMEM((_C,), jnp.float32),
        ],
    )


def _tc_reduce(logits_ref, labels_ref, code_ref, bce_ref, dice_ref):
    i = pl.program_id(0)
    l = logits_ref[...]
    t = labels_ref[...]
    codev = code_ref[...]
    cand = codev == 1.0
    alw = codev == 2.0
    absl = jnp.abs(l)
    bits = lax.bitcast_convert_type(absl, jnp.int32)
    bits = jnp.where(cand, bits, jnp.int32(2**31 - 1))

    def body(_, carry):
        lo, hi = carry
        mid = lo + lax.shift_right_logical(hi - lo, 1)
        cnt = jnp.sum((bits <= mid).astype(jnp.int32))
        ge = cnt >= _K
        return jnp.where(ge, lo, mid + 1), jnp.where(ge, mid, hi)

    _, thr = lax.fori_loop(0, 31, body,
                           (jnp.int32(0), jnp.int32(2**31 - 1)))

    full = jnp.where((cand & (bits <= thr)) | alw, jnp.float32(1.0), jnp.float32(0.0))
    bce = (jnp.maximum(l, 0.0) - l * t + jnp.log1p(jnp.exp(-absl))) * full
    sig = jnp.float32(1.0) / (jnp.float32(1.0) + jnp.exp(-l))
    s1 = jnp.sum(sig * t * full)
    s2 = jnp.sum(sig * full)
    s3 = jnp.sum(t * full)
    dice = jnp.float32(1.0) - (2.0 * s1 + 1.0) / (s2 + s3 + 1.0)

    @pl.when(i == 0)
    def _():
        bce_ref[...] = jnp.zeros_like(bce_ref)
        dice_ref[...] = jnp.zeros_like(dice_ref)

    bce_ref[...] = bce_ref[...] + jnp.sum(bce)
    dice_ref[...] = dice_ref[...] + dice


def kernel(pred, target):
    p = pred.reshape(_N, _H, _W)
    t = target.reshape(_N, _H, _W)
    out_l, out_t = _sc_sample()(p, t, _TBL_BASE.reshape(-1),
                                _TBL_W[0].reshape(-1), _TBL_W[1].reshape(-1),
                                _TBL_W[2].reshape(-1), _TBL_W[3].reshape(-1))
    rpi = _M // 128
    bce_sum, dice_sum = pl.pallas_call(
        _tc_reduce,
        grid=(_N,),
        in_specs=[
            pl.BlockSpec((rpi, 128), lambda i: (i, 0)),
            pl.BlockSpec((rpi, 128), lambda i: (i, 0)),
            pl.BlockSpec((rpi, 128), lambda i: (i, 0)),
        ],
        out_specs=[
            pl.BlockSpec((1, 1), lambda i: (0, 0)),
            pl.BlockSpec((1, 1), lambda i: (0, 0)),
        ],
        out_shape=[
            jax.ShapeDtypeStruct((1, 1), jnp.float32),
            jax.ShapeDtypeStruct((1, 1), jnp.float32),
        ],
    )(out_l.reshape(_N * rpi, 128), out_t.reshape(_N * rpi, 128), _CODEF)
    loss_bce = bce_sum[0, 0] / jnp.float32(_N * _P)
    loss_dice = dice_sum[0, 0] / jnp.float32(_N)
    loss = loss_bce + loss_dice
    return loss, loss_bce, loss_dice


# speedup vs baseline: 1.5753x; 1.5753x over previous
import functools

import numpy as np
import jax
import jax.numpy as jnp
from jax import lax
from jax.experimental import pallas as pl
from jax.experimental.pallas import tpu as pltpu
from jax.experimental.pallas import tpu_sc as plsc

_N, _H, _W = 64, 512, 512
_S = 112 * 112 * 3
_P = 112 * 112
_K = int(0.75 * _P)
_R = _P - _K
_NB = 8
_BAND_ROWS = 72
_ROW0 = np.array([min(b * 64, _H - _BAND_ROWS) for b in range(_NB)], dtype=np.int32)


def _threefry2x32(k1, k2, x0, x1):
    k1 = np.uint32(k1)
    k2 = np.uint32(k2)
    x0 = x0.astype(np.uint32).copy()
    x1 = x1.astype(np.uint32).copy()
    ks = [k1, k2, np.uint32(k1 ^ k2 ^ np.uint32(0x1BD11BDA))]
    rot = [(13, 15, 26, 6), (17, 29, 16, 24)]
    x0 = x0 + ks[0]
    x1 = x1 + ks[1]
    for g in range(5):
        for r in rot[g % 2]:
            x0 = x0 + x1
            x1 = (x1 << np.uint32(r)) | (x1 >> np.uint32(32 - r))
            x1 = x0 ^ x1
        x0 = x0 + ks[(g + 1) % 3]
        x1 = x1 + ks[(g + 2) % 3] + np.uint32(g + 1)
    return x0, x1


def _np_uniform(rawkey, shape):
    size = int(np.prod(shape))
    b1, b2 = _threefry2x32(rawkey[0], rawkey[1],
                           np.zeros(size, np.uint32),
                           np.arange(size, dtype=np.uint32))
    bits = b1 ^ b2
    fb = (bits >> np.uint32(9)) | np.uint32(0x3F800000)
    return (fb.view(np.float32) - np.float32(1.0)).reshape(shape)


def _build_tables():
    b1, b2 = _threefry2x32(np.uint32(0), np.uint32(42),
                           np.zeros(2, np.uint32), np.arange(2, dtype=np.uint32))
    pc = _np_uniform((b1[0], b2[0]), (_N, _S, 2))
    rc = _np_uniform((b1[1], b2[1]), (_N, _R, 2))
    coords = np.concatenate([pc, rc], axis=1)

    x = coords[..., 0] * np.float32(_W) - np.float32(0.5)
    y = coords[..., 1] * np.float32(_H) - np.float32(0.5)
    x0 = np.floor(x)
    y0 = np.floor(y)
    fx1 = x - x0
    fx0 = np.float32(1.0) - fx1
    fy1 = y - y0
    fy0 = np.float32(1.0) - fy1

    def slot_weights(c0, f0, f1, lim):
        base = np.clip(c0, 0.0, lim - 2.0).astype(np.int32)
        g = np.zeros(c0.shape + (2,), dtype=np.float32)
        for d, f in ((0, f0), (1, f1)):
            ic = c0 + d
            valid = (ic >= 0) & (ic <= lim - 1)
            slot = np.clip(ic.astype(np.int64) - base, 0, 1).astype(np.int32)
            for s_ in (0, 1):
                g[..., s_] += np.where(valid & (slot == s_), f, np.float32(0.0))
        return base, g

    basex, gx = slot_weights(x0, fx0, fx1, _W)
    basey, gy = slot_weights(y0, fy0, fy1, _H)

    band = np.minimum(basey >> 6, _NB - 1).astype(np.int32)
    base_local = (basey - _ROW0[band]) * _W + basex
    ws = (gx[..., 0] * gy[..., 0], gx[..., 1] * gy[..., 0],
          gx[..., 0] * gy[..., 1], gx[..., 1] * gy[..., 1])

    counts = np.zeros((_N, _NB), np.int32)
    for n in range(_N):
        counts[n] = np.bincount(band[n], minlength=_NB)
    cap = int(counts.max())
    cap = (cap + 127) // 128 * 128

    tbl_base = np.zeros((_N, _NB, cap), np.int32)
    tbl_w = np.zeros((4, _N, _NB, cap), np.float32)
    code = np.zeros((_N, _NB, cap), np.float32)
    for n in range(_N):
        for b in range(_NB):
            i1 = np.nonzero(band[n, :_S] == b)[0]
            i2 = np.nonzero(band[n, _S:] == b)[0] + _S
            idx = np.concatenate([i1, i2])
            c = len(idx)
            tbl_base[n, b, :c] = base_local[n, idx]
            for k_ in range(4):
                tbl_w[k_, n, b, :c] = ws[k_][n, idx]
            code[n, b, :len(i1)] = 1.0
            code[n, b, len(i1):c] = 2.0
    return tbl_base, tbl_w, code, cap


_TBL_BASE, _TBL_W, _CODE, _C = _build_tables()
_M = _NB * _C
_CODEF = _CODE.reshape(_N, _M // 128, 128)

_TASKS_PER_TILE = (_N * _NB) // 32


_BAND_WORDS = _BAND_ROWS * _W


def _sc_sample_body(pred_hbm, target_hbm, base_hbm, w00_hbm, w01_hbm, w10_hbm,
                    w11_hbm, out_l_hbm, out_t_hbm,
                    band_p, band_t, base_v, w00_v, w01_v, w10_v, w11_v,
                    out_lv, out_tv):
    wid = lax.axis_index("s") * 2 + lax.axis_index("c")

    def task(t, carry):
        gid = wid * _TASKS_PER_TILE + t
        n = gid // _NB
        b = gid % _NB
        row0 = jnp.minimum(b * 64, _H - _BAND_ROWS)
        tbl_off = gid * _C
        pltpu.sync_copy(pred_hbm.at[n, pl.ds(row0, _BAND_ROWS), :], band_p)
        pltpu.sync_copy(target_hbm.at[n, pl.ds(row0, _BAND_ROWS), :], band_t)
        pltpu.sync_copy(base_hbm.at[pl.ds(tbl_off, _C)], base_v)
        pltpu.sync_copy(w00_hbm.at[pl.ds(tbl_off, _C)], w00_v)
        pltpu.sync_copy(w01_hbm.at[pl.ds(tbl_off, _C)], w01_v)
        pltpu.sync_copy(w10_hbm.at[pl.ds(tbl_off, _C)], w10_v)
        pltpu.sync_copy(w11_hbm.at[pl.ds(tbl_off, _C)], w11_v)

        def group(j, c2):
            sl = pl.ds(j * 16, 16)
            i00 = base_v[sl]
            by = lax.shift_right_logical(i00, 9)
            bx = lax.bitwise_and(i00, 511)
            by1 = by + 1
            bx1 = bx + 1
            a00 = w00_v[sl]
            a01 = w01_v[sl]
            a10 = w10_v[sl]
            a11 = w11_v[sl]
            out_lv[sl] = (plsc.load_gather(band_p, [by, bx]) * a00
                          + plsc.load_gather(band_p, [by, bx1]) * a01
                          + plsc.load_gather(band_p, [by1, bx]) * a10
                          + plsc.load_gather(band_p, [by1, bx1]) * a11)
            out_tv[sl] = (plsc.load_gather(band_t, [by, bx]) * a00
                          + plsc.load_gather(band_t, [by, bx1]) * a01
                          + plsc.load_gather(band_t, [by1, bx]) * a10
                          + plsc.load_gather(band_t, [by1, bx1]) * a11)
            return c2

        lax.fori_loop(0, _C // 16, group, 0)
        pltpu.sync_copy(out_lv, out_l_hbm.at[pl.ds(tbl_off, _C)])
        pltpu.sync_copy(out_tv, out_t_hbm.at[pl.ds(tbl_off, _C)])
        return carry

    lax.fori_loop(0, _TASKS_PER_TILE, task, 0)


@functools.cache
def _sc_sample():
    mesh = plsc.VectorSubcoreMesh(core_axis_name="c", subcore_axis_name="s",
                                  num_cores=2, num_subcores=16)
    return pl.kernel(
        _sc_sample_body,
        out_type=(jax.ShapeDtypeStruct((_N * _NB * _C,), jnp.float32),
                  jax.ShapeDtypeStruct((_N * _NB * _C,), jnp.float32)),
        mesh=mesh,
        compiler_params=pltpu.CompilerParams(needs_layout_passes=False),
        scratch_types=[
            pltpu.VMEM((_BAND_ROWS, _W), jnp.float32),
            pltpu.VMEM((_BAND_ROWS, _W), jnp.float32),
            pltpu.VMEM((_C,), jnp.int32),
            pltpu.VMEM((_C,), jnp.float32),
            pltpu.VMEM((_C,), jnp.float32),
            pltpu.VMEM((_C,), jnp.float32),
            pltpu.VMEM((_C,), jnp.float32),
            pltpu.VMEM((_C,), jnp.float32),
            pltpu.VMEM((_C,), jnp.float32),
        ],
    )


_IB = 8


def _tc_reduce(logits_ref, labels_ref, code_ref, bce_ref, dice_ref):
    i = pl.program_id(0)
    l = logits_ref[...]
    t = labels_ref[...]
    codev = code_ref[...]
    cand = codev == 1.0
    alw = codev == 2.0
    absl = jnp.abs(l)
    bits = lax.bitcast_convert_type(absl, jnp.int32)
    bits = jnp.where(cand, bits, jnp.int32(2**31 - 1))

    def body(_, carry):
        lo, hi = carry
        mid = lo + lax.shift_right_logical(hi - lo, 1)
        cnt = jnp.sum((bits <= mid).astype(jnp.int32), axis=(1, 2), keepdims=True)
        ge = cnt >= _K
        return jnp.where(ge, lo, mid + 1), jnp.where(ge, mid, hi)

    lo0 = jnp.zeros((_IB, 1, 1), jnp.int32)
    hi0 = jnp.full((_IB, 1, 1), 2**31 - 1, jnp.int32)
    _, thr = lax.fori_loop(0, 31, body, (lo0, hi0))

    full = jnp.where((cand & (bits <= thr)) | alw, jnp.float32(1.0), jnp.float32(0.0))
    bce = (jnp.maximum(l, 0.0) - l * t + jnp.log1p(jnp.exp(-absl))) * full
    sig = jnp.float32(1.0) / (jnp.float32(1.0) + jnp.exp(-l))
    s1 = jnp.sum(sig * t * full, axis=(1, 2))
    s2 = jnp.sum(sig * full, axis=(1, 2))
    s3 = jnp.sum(t * full, axis=(1, 2))
    dice = jnp.float32(1.0) - (2.0 * s1 + 1.0) / (s2 + s3 + 1.0)

    @pl.when(i == 0)
    def _():
        bce_ref[...] = jnp.zeros_like(bce_ref)
        dice_ref[...] = jnp.zeros_like(dice_ref)

    bce_ref[...] = bce_ref[...] + jnp.sum(bce)
    dice_ref[...] = dice_ref[...] + jnp.sum(dice)


def kernel(pred, target):
    p = pred.reshape(_N, _H, _W)
    t = target.reshape(_N, _H, _W)
    out_l, out_t = _sc_sample()(p, t, _TBL_BASE.reshape(-1),
                                _TBL_W[0].reshape(-1), _TBL_W[1].reshape(-1),
                                _TBL_W[2].reshape(-1), _TBL_W[3].reshape(-1))
    rpi = _M // 128
    bce_sum, dice_sum = pl.pallas_call(
        _tc_reduce,
        grid=(_N // _IB,),
        in_specs=[
            pl.BlockSpec((_IB, rpi, 128), lambda i: (i, 0, 0)),
            pl.BlockSpec((_IB, rpi, 128), lambda i: (i, 0, 0)),
            pl.BlockSpec((_IB, rpi, 128), lambda i: (i, 0, 0)),
        ],
        out_specs=[
            pl.BlockSpec((1, 1), lambda i: (0, 0)),
            pl.BlockSpec((1, 1), lambda i: (0, 0)),
        ],
        out_shape=[
            jax.ShapeDtypeStruct((1, 1), jnp.float32),
            jax.ShapeDtypeStruct((1, 1), jnp.float32),
        ],
    )(out_l.reshape(_N, rpi, 128), out_t.reshape(_N, rpi, 128), _CODEF)
    loss_bce = bce_sum[0, 0] / jnp.float32(_N * _P)
    loss_dice = dice_sum[0, 0] / jnp.float32(_N)
    loss = loss_bce + loss_dice
    return loss, loss_bce, loss_dice


# speedup vs baseline: 2.1366x; 1.3563x over previous
import functools

import numpy as np
import jax
import jax.numpy as jnp
from jax import lax
from jax.experimental import pallas as pl
from jax.experimental.pallas import tpu as pltpu
from jax.experimental.pallas import tpu_sc as plsc

_N, _H, _W = 64, 512, 512
_S = 112 * 112 * 3
_P = 112 * 112
_K = int(0.75 * _P)
_R = _P - _K
_NB = 8
_BAND_ROWS = 72
_ROW0 = np.array([min(b * 64, _H - _BAND_ROWS) for b in range(_NB)], dtype=np.int32)


def _threefry2x32(k1, k2, x0, x1):
    k1 = np.uint32(k1)
    k2 = np.uint32(k2)
    x0 = x0.astype(np.uint32).copy()
    x1 = x1.astype(np.uint32).copy()
    ks = [k1, k2, np.uint32(k1 ^ k2 ^ np.uint32(0x1BD11BDA))]
    rot = [(13, 15, 26, 6), (17, 29, 16, 24)]
    x0 = x0 + ks[0]
    x1 = x1 + ks[1]
    for g in range(5):
        for r in rot[g % 2]:
            x0 = x0 + x1
            x1 = (x1 << np.uint32(r)) | (x1 >> np.uint32(32 - r))
            x1 = x0 ^ x1
        x0 = x0 + ks[(g + 1) % 3]
        x1 = x1 + ks[(g + 2) % 3] + np.uint32(g + 1)
    return x0, x1


def _np_uniform(rawkey, shape):
    size = int(np.prod(shape))
    b1, b2 = _threefry2x32(rawkey[0], rawkey[1],
                           np.zeros(size, np.uint32),
                           np.arange(size, dtype=np.uint32))
    bits = b1 ^ b2
    fb = (bits >> np.uint32(9)) | np.uint32(0x3F800000)
    return (fb.view(np.float32) - np.float32(1.0)).reshape(shape)


def _build_tables():
    b1, b2 = _threefry2x32(np.uint32(0), np.uint32(42),
                           np.zeros(2, np.uint32), np.arange(2, dtype=np.uint32))
    pc = _np_uniform((b1[0], b2[0]), (_N, _S, 2))
    rc = _np_uniform((b1[1], b2[1]), (_N, _R, 2))
    coords = np.concatenate([pc, rc], axis=1)

    x = coords[..., 0] * np.float32(_W) - np.float32(0.5)
    y = coords[..., 1] * np.float32(_H) - np.float32(0.5)
    x0 = np.floor(x)
    y0 = np.floor(y)
    fx1 = x - x0
    fx0 = np.float32(1.0) - fx1
    fy1 = y - y0
    fy0 = np.float32(1.0) - fy1

    def slot_weights(c0, f0, f1, lim):
        base = np.clip(c0, 0.0, lim - 2.0).astype(np.int32)
        g = np.zeros(c0.shape + (2,), dtype=np.float32)
        for d, f in ((0, f0), (1, f1)):
            ic = c0 + d
            valid = (ic >= 0) & (ic <= lim - 1)
            slot = np.clip(ic.astype(np.int64) - base, 0, 1).astype(np.int32)
            for s_ in (0, 1):
                g[..., s_] += np.where(valid & (slot == s_), f, np.float32(0.0))
        return base, g

    basex, gx = slot_weights(x0, fx0, fx1, _W)
    basey, gy = slot_weights(y0, fy0, fy1, _H)

    band = np.minimum(basey >> 6, _NB - 1).astype(np.int32)
    base_local = (basey - _ROW0[band]) * _W + basex
    ws = (gx[..., 0] * gy[..., 0], gx[..., 1] * gy[..., 0],
          gx[..., 0] * gy[..., 1], gx[..., 1] * gy[..., 1])

    counts = np.zeros((_N, _NB), np.int32)
    for n in range(_N):
        counts[n] = np.bincount(band[n], minlength=_NB)
    cap = int(counts.max())
    cap = (cap + 127) // 128 * 128

    tbl_base = np.zeros((_N, _NB, cap), np.int32)
    tbl_w = np.zeros((4, _N, _NB, cap), np.float32)
    code = np.zeros((_N, _NB, cap), np.float32)
    for n in range(_N):
        for b in range(_NB):
            i1 = np.nonzero(band[n, :_S] == b)[0]
            i2 = np.nonzero(band[n, _S:] == b)[0] + _S
            idx = np.concatenate([i1, i2])
            c = len(idx)
            tbl_base[n, b, :c] = base_local[n, idx]
            for k_ in range(4):
                tbl_w[k_, n, b, :c] = ws[k_][n, idx]
            code[n, b, :len(i1)] = 1.0
            code[n, b, len(i1):c] = 2.0
    return tbl_base, tbl_w, code, cap


_TBL_BASE, _TBL_W, _CODE, _C = _build_tables()
_M = _NB * _C
_CODEF = _CODE.reshape(_N, _M // 128, 128)

_TASKS_PER_TILE = (_N * _NB) // 32


_BAND_WORDS = _BAND_ROWS * _W


def _sc_sample_body(pred_hbm, target_hbm, base_hbm, w00_hbm, w01_hbm, w10_hbm,
                    w11_hbm, out_l_hbm, out_t_hbm,
                    band_p, band_t, base_v, w00_v, w01_v, w10_v, w11_v,
                    out_lv, out_tv, sem):
    wid = lax.axis_index("s") * 2 + lax.axis_index("c")

    def task(t, carry):
        gid = wid * _TASKS_PER_TILE + t
        n = gid // _NB
        b = gid % _NB
        row0 = jnp.minimum(b * 64, _H - _BAND_ROWS)
        tbl_off = gid * _C
        cps = [
            pltpu.async_copy(pred_hbm.at[n, pl.ds(row0, _BAND_ROWS), :],
                             band_p, sem),
            pltpu.async_copy(target_hbm.at[n, pl.ds(row0, _BAND_ROWS), :],
                             band_t, sem),
            pltpu.async_copy(base_hbm.at[pl.ds(tbl_off, _C)], base_v, sem),
            pltpu.async_copy(w00_hbm.at[pl.ds(tbl_off, _C)], w00_v, sem),
            pltpu.async_copy(w01_hbm.at[pl.ds(tbl_off, _C)], w01_v, sem),
            pltpu.async_copy(w10_hbm.at[pl.ds(tbl_off, _C)], w10_v, sem),
            pltpu.async_copy(w11_hbm.at[pl.ds(tbl_off, _C)], w11_v, sem),
        ]
        for cp in cps:
            cp.wait()

        @plsc.parallel_loop(0, _C // 16, unroll=4)
        def group(j):
            sl = pl.ds(j * 16, 16)
            i00 = base_v[sl]
            by = lax.shift_right_logical(i00, 9)
            bx = lax.bitwise_and(i00, 511)
            by1 = by + 1
            bx1 = bx + 1
            a00 = w00_v[sl]
            a01 = w01_v[sl]
            a10 = w10_v[sl]
            a11 = w11_v[sl]
            out_lv[sl] = (plsc.load_gather(band_p, [by, bx]) * a00
                          + plsc.load_gather(band_p, [by, bx1]) * a01
                          + plsc.load_gather(band_p, [by1, bx]) * a10
                          + plsc.load_gather(band_p, [by1, bx1]) * a11)
            out_tv[sl] = (plsc.load_gather(band_t, [by, bx]) * a00
                          + plsc.load_gather(band_t, [by, bx1]) * a01
                          + plsc.load_gather(band_t, [by1, bx]) * a10
                          + plsc.load_gather(band_t, [by1, bx1]) * a11)

        pltpu.sync_copy(out_lv, out_l_hbm.at[pl.ds(tbl_off, _C)])
        pltpu.sync_copy(out_tv, out_t_hbm.at[pl.ds(tbl_off, _C)])
        return carry

    lax.fori_loop(0, _TASKS_PER_TILE, task, 0)


@functools.cache
def _sc_sample():
    mesh = plsc.VectorSubcoreMesh(core_axis_name="c", subcore_axis_name="s",
                                  num_cores=2, num_subcores=16)
    return pl.kernel(
        _sc_sample_body,
        out_type=(jax.ShapeDtypeStruct((_N * _NB * _C,), jnp.float32),
                  jax.ShapeDtypeStruct((_N * _NB * _C,), jnp.float32)),
        mesh=mesh,
        compiler_params=pltpu.CompilerParams(needs_layout_passes=False),
        scratch_types=[
            pltpu.VMEM((_BAND_ROWS, _W), jnp.float32),
            pltpu.VMEM((_BAND_ROWS, _W), jnp.float32),
            pltpu.VMEM((_C,), jnp.int32),
            pltpu.VMEM((_C,), jnp.float32),
            pltpu.VMEM((_C,), jnp.float32),
            pltpu.VMEM((_C,), jnp.float32),
            pltpu.VMEM((_C,), jnp.float32),
            pltpu.VMEM((_C,), jnp.float32),
            pltpu.VMEM((_C,), jnp.float32),
            pltpu.SemaphoreType.DMA,
        ],
    )


_IB = 8


def _tc_reduce(logits_ref, labels_ref, code_ref, bce_ref, dice_ref):
    i = pl.program_id(0)
    l = logits_ref[...]
    t = labels_ref[...]
    codev = code_ref[...]
    cand = codev == 1.0
    alw = codev == 2.0
    absl = jnp.abs(l)
    bits = lax.bitcast_convert_type(absl, jnp.int32)
    bits = jnp.where(cand, bits, jnp.int32(2**31 - 1))

    def body(_, carry):
        lo, hi = carry
        mid = lo + lax.shift_right_logical(hi - lo, 1)
        cnt = jnp.sum((bits <= mid).astype(jnp.int32), axis=(1, 2), keepdims=True)
        ge = cnt >= _K
        return jnp.where(ge, lo, mid + 1), jnp.where(ge, mid, hi)

    lo0 = jnp.zeros((_IB, 1, 1), jnp.int32)
    hi0 = jnp.full((_IB, 1, 1), 2**31 - 1, jnp.int32)
    _, thr = lax.fori_loop(0, 31, body, (lo0, hi0))

    full = jnp.where((cand & (bits <= thr)) | alw, jnp.float32(1.0), jnp.float32(0.0))
    bce = (jnp.maximum(l, 0.0) - l * t + jnp.log1p(jnp.exp(-absl))) * full
    sig = jnp.float32(1.0) / (jnp.float32(1.0) + jnp.exp(-l))
    s1 = jnp.sum(sig * t * full, axis=(1, 2))
    s2 = jnp.sum(sig * full, axis=(1, 2))
    s3 = jnp.sum(t * full, axis=(1, 2))
    dice = jnp.float32(1.0) - (2.0 * s1 + 1.0) / (s2 + s3 + 1.0)

    @pl.when(i == 0)
    def _():
        bce_ref[...] = jnp.zeros_like(bce_ref)
        dice_ref[...] = jnp.zeros_like(dice_ref)

    bce_ref[...] = bce_ref[...] + jnp.sum(bce)
    dice_ref[...] = dice_ref[...] + jnp.sum(dice)


def kernel(pred, target):
    p = pred.reshape(_N, _H, _W)
    t = target.reshape(_N, _H, _W)
    out_l, out_t = _sc_sample()(p, t, _TBL_BASE.reshape(-1),
                                _TBL_W[0].reshape(-1), _TBL_W[1].reshape(-1),
                                _TBL_W[2].reshape(-1), _TBL_W[3].reshape(-1))
    rpi = _M // 128
    bce_sum, dice_sum = pl.pallas_call(
        _tc_reduce,
        grid=(_N // _IB,),
        in_specs=[
            pl.BlockSpec((_IB, rpi, 128), lambda i: (i, 0, 0)),
            pl.BlockSpec((_IB, rpi, 128), lambda i: (i, 0, 0)),
            pl.BlockSpec((_IB, rpi, 128), lambda i: (i, 0, 0)),
        ],
        out_specs=[
            pl.BlockSpec((1, 1), lambda i: (0, 0)),
            pl.BlockSpec((1, 1), lambda i: (0, 0)),
        ],
        out_shape=[
            jax.ShapeDtypeStruct((1, 1), jnp.float32),
            jax.ShapeDtypeStruct((1, 1), jnp.float32),
        ],
    )(out_l.reshape(_N, rpi, 128), out_t.reshape(_N, rpi, 128), _CODEF)
    loss_bce = bce_sum[0, 0] / jnp.float32(_N * _P)
    loss_dice = dice_sum[0, 0] / jnp.float32(_N)
    loss = loss_bce + loss_dice
    return loss, loss_bce, loss_dice


# speedup vs baseline: 2.2662x; 1.0607x over previous
import functools

import numpy as np
import jax
import jax.numpy as jnp
from jax import lax
from jax.experimental import pallas as pl
from jax.experimental.pallas import tpu as pltpu
from jax.experimental.pallas import tpu_sc as plsc

_N, _H, _W = 64, 512, 512
_S = 112 * 112 * 3
_P = 112 * 112
_K = int(0.75 * _P)
_R = _P - _K
_NB = 8
_BAND_ROWS = 72
_ROW0 = np.array([min(b * 64, _H - _BAND_ROWS) for b in range(_NB)], dtype=np.int32)


def _threefry2x32(k1, k2, x0, x1):
    k1 = np.uint32(k1)
    k2 = np.uint32(k2)
    x0 = x0.astype(np.uint32).copy()
    x1 = x1.astype(np.uint32).copy()
    ks = [k1, k2, np.uint32(k1 ^ k2 ^ np.uint32(0x1BD11BDA))]
    rot = [(13, 15, 26, 6), (17, 29, 16, 24)]
    x0 = x0 + ks[0]
    x1 = x1 + ks[1]
    for g in range(5):
        for r in rot[g % 2]:
            x0 = x0 + x1
            x1 = (x1 << np.uint32(r)) | (x1 >> np.uint32(32 - r))
            x1 = x0 ^ x1
        x0 = x0 + ks[(g + 1) % 3]
        x1 = x1 + ks[(g + 2) % 3] + np.uint32(g + 1)
    return x0, x1


def _np_uniform(rawkey, shape):
    size = int(np.prod(shape))
    b1, b2 = _threefry2x32(rawkey[0], rawkey[1],
                           np.zeros(size, np.uint32),
                           np.arange(size, dtype=np.uint32))
    bits = b1 ^ b2
    fb = (bits >> np.uint32(9)) | np.uint32(0x3F800000)
    return (fb.view(np.float32) - np.float32(1.0)).reshape(shape)


def _build_tables():
    b1, b2 = _threefry2x32(np.uint32(0), np.uint32(42),
                           np.zeros(2, np.uint32), np.arange(2, dtype=np.uint32))
    pc = _np_uniform((b1[0], b2[0]), (_N, _S, 2))
    rc = _np_uniform((b1[1], b2[1]), (_N, _R, 2))
    coords = np.concatenate([pc, rc], axis=1)

    x = coords[..., 0] * np.float32(_W) - np.float32(0.5)
    y = coords[..., 1] * np.float32(_H) - np.float32(0.5)
    x0 = np.floor(x)
    y0 = np.floor(y)
    fx1 = x - x0
    fx0 = np.float32(1.0) - fx1
    fy1 = y - y0
    fy0 = np.float32(1.0) - fy1

    def slot_weights(c0, f0, f1, lim):
        base = np.clip(c0, 0.0, lim - 2.0).astype(np.int32)
        g = np.zeros(c0.shape + (2,), dtype=np.float32)
        for d, f in ((0, f0), (1, f1)):
            ic = c0 + d
            valid = (ic >= 0) & (ic <= lim - 1)
            slot = np.clip(ic.astype(np.int64) - base, 0, 1).astype(np.int32)
            for s_ in (0, 1):
                g[..., s_] += np.where(valid & (slot == s_), f, np.float32(0.0))
        return base, g

    basex, gx = slot_weights(x0, fx0, fx1, _W)
    basey, gy = slot_weights(y0, fy0, fy1, _H)

    band = np.minimum(basey >> 6, _NB - 1).astype(np.int32)
    base_local = (basey - _ROW0[band]) * _W + basex
    ws = (gx[..., 0] * gy[..., 0], gx[..., 1] * gy[..., 0],
          gx[..., 0] * gy[..., 1], gx[..., 1] * gy[..., 1])

    counts = np.zeros((_N, _NB), np.int32)
    for n in range(_N):
        counts[n] = np.bincount(band[n], minlength=_NB)
    cap = int(counts.max())
    cap = (cap + 127) // 128 * 128

    tbl_base = np.zeros((_N, _NB, cap), np.int32)
    tbl_w = np.zeros((4, _N, _NB, cap), np.float32)
    code = np.zeros((_N, _NB, cap), np.float32)
    for n in range(_N):
        for b in range(_NB):
            i1 = np.nonzero(band[n, :_S] == b)[0]
            i2 = np.nonzero(band[n, _S:] == b)[0] + _S
            idx = np.concatenate([i1, i2])
            c = len(idx)
            tbl_base[n, b, :c] = base_local[n, idx]
            for k_ in range(4):
                tbl_w[k_, n, b, :c] = ws[k_][n, idx]
            code[n, b, :len(i1)] = 1.0
            code[n, b, len(i1):c] = 2.0
    return tbl_base, tbl_w, code, cap


_TBL_BASE, _TBL_W, _CODE, _C = _build_tables()
_M = _NB * _C
_CODEF = _CODE.reshape(_N, _M // 128, 128)

_TASKS_PER_TILE = (_N * _NB) // 32


_BAND_WORDS = _BAND_ROWS * _W


def _sc_sample_body(pred_hbm, target_hbm, base_hbm, w00_hbm, w01_hbm, w10_hbm,
                    w11_hbm, out_l_hbm, out_t_hbm,
                    band_p, band_t, base_v, w00_v, w01_v, w10_v, w11_v,
                    out_lv, out_tv, sem):
    wid = lax.axis_index("s") * 2 + lax.axis_index("c")

    def task(t, carry):
        gid = wid * _TASKS_PER_TILE + t
        n = gid // _NB
        b = gid % _NB
        row0 = jnp.minimum(b * 64, _H - _BAND_ROWS)
        tbl_off = gid * _C
        cps = [
            pltpu.async_copy(pred_hbm.at[n, pl.ds(row0, _BAND_ROWS), :],
                             band_p, sem),
            pltpu.async_copy(target_hbm.at[n, pl.ds(row0, _BAND_ROWS), :],
                             band_t, sem),
            pltpu.async_copy(base_hbm.at[pl.ds(tbl_off, _C)], base_v, sem),
            pltpu.async_copy(w00_hbm.at[pl.ds(tbl_off, _C)], w00_v, sem),
            pltpu.async_copy(w01_hbm.at[pl.ds(tbl_off, _C)], w01_v, sem),
            pltpu.async_copy(w10_hbm.at[pl.ds(tbl_off, _C)], w10_v, sem),
            pltpu.async_copy(w11_hbm.at[pl.ds(tbl_off, _C)], w11_v, sem),
        ]
        for cp in cps:
            cp.wait()

        @plsc.parallel_loop(0, _C // 16, unroll=4)
        def group(j):
            sl = pl.ds(j * 16, 16)
            i00 = base_v[sl]
            by = lax.shift_right_logical(i00, 9)
            bx = lax.bitwise_and(i00, 511)
            by1 = by + 1
            bx1 = bx + 1
            a00 = w00_v[sl]
            a01 = w01_v[sl]
            a10 = w10_v[sl]
            a11 = w11_v[sl]
            out_lv[sl] = (plsc.load_gather(band_p, [by, bx]) * a00
                          + plsc.load_gather(band_p, [by, bx1]) * a01
                          + plsc.load_gather(band_p, [by1, bx]) * a10
                          + plsc.load_gather(band_p, [by1, bx1]) * a11)
            out_tv[sl] = (plsc.load_gather(band_t, [by, bx]) * a00
                          + plsc.load_gather(band_t, [by, bx1]) * a01
                          + plsc.load_gather(band_t, [by1, bx]) * a10
                          + plsc.load_gather(band_t, [by1, bx1]) * a11)

        pltpu.sync_copy(out_lv, out_l_hbm.at[pl.ds(tbl_off, _C)])
        pltpu.sync_copy(out_tv, out_t_hbm.at[pl.ds(tbl_off, _C)])
        return carry

    lax.fori_loop(0, _TASKS_PER_TILE, task, 0)


@functools.cache
def _sc_sample():
    mesh = plsc.VectorSubcoreMesh(core_axis_name="c", subcore_axis_name="s",
                                  num_cores=2, num_subcores=16)
    return pl.kernel(
        _sc_sample_body,
        out_type=(jax.ShapeDtypeStruct((_N * _NB * _C,), jnp.float32),
                  jax.ShapeDtypeStruct((_N * _NB * _C,), jnp.float32)),
        mesh=mesh,
        compiler_params=pltpu.CompilerParams(needs_layout_passes=False),
        scratch_types=[
            pltpu.VMEM((_BAND_ROWS, _W), jnp.float32),
            pltpu.VMEM((_BAND_ROWS, _W), jnp.float32),
            pltpu.VMEM((_C,), jnp.int32),
            pltpu.VMEM((_C,), jnp.float32),
            pltpu.VMEM((_C,), jnp.float32),
            pltpu.VMEM((_C,), jnp.float32),
            pltpu.VMEM((_C,), jnp.float32),
            pltpu.VMEM((_C,), jnp.float32),
            pltpu.VMEM((_C,), jnp.float32),
            pltpu.SemaphoreType.DMA,
        ],
    )


_IB = 32


def _tc_reduce(logits_ref, labels_ref, code_ref, bce_ref, dice_ref):
    i = pl.program_id(0)
    l = logits_ref[...]
    t = labels_ref[...]
    codev = code_ref[...]
    cand = codev == 1.0
    alw = codev == 2.0
    absl = jnp.abs(l)
    bits = lax.bitcast_convert_type(absl, jnp.int32)
    bits = jnp.where(cand, bits, jnp.int32(2**31 - 1))

    def body(_, carry):
        lo, hi = carry
        mid = lo + lax.shift_right_logical(hi - lo, 1)
        cnt = jnp.sum((bits <= mid).astype(jnp.int32), axis=(1, 2), keepdims=True)
        ge = cnt >= _K
        return jnp.where(ge, lo, mid + 1), jnp.where(ge, mid, hi)

    lo0 = jnp.zeros((_IB, 1, 1), jnp.int32)
    hi0 = jnp.full((_IB, 1, 1), 2**31 - 1, jnp.int32)
    _, thr = lax.fori_loop(0, 31, body, (lo0, hi0))

    full = jnp.where((cand & (bits <= thr)) | alw, jnp.float32(1.0), jnp.float32(0.0))
    bce = (jnp.maximum(l, 0.0) - l * t + jnp.log1p(jnp.exp(-absl))) * full
    sig = jnp.float32(1.0) / (jnp.float32(1.0) + jnp.exp(-l))
    s1 = jnp.sum(sig * t * full, axis=(1, 2))
    s2 = jnp.sum(sig * full, axis=(1, 2))
    s3 = jnp.sum(t * full, axis=(1, 2))
    dice = jnp.float32(1.0) - (2.0 * s1 + 1.0) / (s2 + s3 + 1.0)

    @pl.when(i == 0)
    def _():
        bce_ref[...] = jnp.zeros_like(bce_ref)
        dice_ref[...] = jnp.zeros_like(dice_ref)

    bce_ref[...] = bce_ref[...] + jnp.sum(bce)
    dice_ref[...] = dice_ref[...] + jnp.sum(dice)


def kernel(pred, target):
    p = pred.reshape(_N, _H, _W)
    t = target.reshape(_N, _H, _W)
    out_l, out_t = _sc_sample()(p, t, _TBL_BASE.reshape(-1),
                                _TBL_W[0].reshape(-1), _TBL_W[1].reshape(-1),
                                _TBL_W[2].reshape(-1), _TBL_W[3].reshape(-1))
    rpi = _M // 128
    bce_sum, dice_sum = pl.pallas_call(
        _tc_reduce,
        grid=(_N // _IB,),
        in_specs=[
            pl.BlockSpec((_IB, rpi, 128), lambda i: (i, 0, 0)),
            pl.BlockSpec((_IB, rpi, 128), lambda i: (i, 0, 0)),
            pl.BlockSpec((_IB, rpi, 128), lambda i: (i, 0, 0)),
        ],
        out_specs=[
            pl.BlockSpec((1, 1), lambda i: (0, 0)),
            pl.BlockSpec((1, 1), lambda i: (0, 0)),
        ],
        out_shape=[
            jax.ShapeDtypeStruct((1, 1), jnp.float32),
            jax.ShapeDtypeStruct((1, 1), jnp.float32),
        ],
    )(out_l.reshape(_N, rpi, 128), out_t.reshape(_N, rpi, 128), _CODEF)
    loss_bce = bce_sum[0, 0] / jnp.float32(_N * _P)
    loss_dice = dice_sum[0, 0] / jnp.float32(_N)
    loss = loss_bce + loss_dice
    return loss, loss_bce, loss_dice


# speedup vs baseline: 2.3230x; 1.0251x over previous
import functools

import numpy as np
import jax
import jax.numpy as jnp
from jax import lax
from jax.experimental import pallas as pl
from jax.experimental.pallas import tpu as pltpu
from jax.experimental.pallas import tpu_sc as plsc

_N, _H, _W = 64, 512, 512
_S = 112 * 112 * 3
_P = 112 * 112
_K = int(0.75 * _P)
_R = _P - _K
_NB = 8
_BAND_ROWS = 72
_ROW0 = np.array([min(b * 64, _H - _BAND_ROWS) for b in range(_NB)], dtype=np.int32)


def _threefry2x32(k1, k2, x0, x1):
    k1 = np.uint32(k1)
    k2 = np.uint32(k2)
    x0 = x0.astype(np.uint32).copy()
    x1 = x1.astype(np.uint32).copy()
    ks = [k1, k2, np.uint32(k1 ^ k2 ^ np.uint32(0x1BD11BDA))]
    rot = [(13, 15, 26, 6), (17, 29, 16, 24)]
    x0 = x0 + ks[0]
    x1 = x1 + ks[1]
    for g in range(5):
        for r in rot[g % 2]:
            x0 = x0 + x1
            x1 = (x1 << np.uint32(r)) | (x1 >> np.uint32(32 - r))
            x1 = x0 ^ x1
        x0 = x0 + ks[(g + 1) % 3]
        x1 = x1 + ks[(g + 2) % 3] + np.uint32(g + 1)
    return x0, x1


def _np_uniform(rawkey, shape):
    size = int(np.prod(shape))
    b1, b2 = _threefry2x32(rawkey[0], rawkey[1],
                           np.zeros(size, np.uint32),
                           np.arange(size, dtype=np.uint32))
    bits = b1 ^ b2
    fb = (bits >> np.uint32(9)) | np.uint32(0x3F800000)
    return (fb.view(np.float32) - np.float32(1.0)).reshape(shape)


def _build_tables():
    b1, b2 = _threefry2x32(np.uint32(0), np.uint32(42),
                           np.zeros(2, np.uint32), np.arange(2, dtype=np.uint32))
    pc = _np_uniform((b1[0], b2[0]), (_N, _S, 2))
    rc = _np_uniform((b1[1], b2[1]), (_N, _R, 2))
    coords = np.concatenate([pc, rc], axis=1)

    x = coords[..., 0] * np.float32(_W) - np.float32(0.5)
    y = coords[..., 1] * np.float32(_H) - np.float32(0.5)
    x0 = np.floor(x)
    y0 = np.floor(y)
    fx1 = x - x0
    fx0 = np.float32(1.0) - fx1
    fy1 = y - y0
    fy0 = np.float32(1.0) - fy1

    def slot_weights(c0, f0, f1, lim):
        base = np.clip(c0, 0.0, lim - 2.0).astype(np.int32)
        g = np.zeros(c0.shape + (2,), dtype=np.float32)
        for d, f in ((0, f0), (1, f1)):
            ic = c0 + d
            valid = (ic >= 0) & (ic <= lim - 1)
            slot = np.clip(ic.astype(np.int64) - base, 0, 1).astype(np.int32)
            for s_ in (0, 1):
                g[..., s_] += np.where(valid & (slot == s_), f, np.float32(0.0))
        return base, g

    basex, gx = slot_weights(x0, fx0, fx1, _W)
    basey, gy = slot_weights(y0, fy0, fy1, _H)

    band = np.minimum(basey >> 6, _NB - 1).astype(np.int32)
    base_local = (basey - _ROW0[band]) * _W + basex
    ws = (gx[..., 0] * gy[..., 0], gx[..., 1] * gy[..., 0],
          gx[..., 0] * gy[..., 1], gx[..., 1] * gy[..., 1])

    counts = np.zeros((_N, _NB), np.int32)
    for n in range(_N):
        counts[n] = np.bincount(band[n], minlength=_NB)
    cap = int(counts.max())
    cap = (cap + 127) // 128 * 128

    tbl_base = np.zeros((_N, _NB, cap), np.int32)
    tbl_w = np.zeros((4, _N, _NB, cap), np.float32)
    code = np.zeros((_N, _NB, cap), np.float32)
    for n in range(_N):
        for b in range(_NB):
            i1 = np.nonzero(band[n, :_S] == b)[0]
            i2 = np.nonzero(band[n, _S:] == b)[0] + _S
            idx = np.concatenate([i1, i2])
            c = len(idx)
            tbl_base[n, b, :c] = base_local[n, idx]
            for k_ in range(4):
                tbl_w[k_, n, b, :c] = ws[k_][n, idx]
            code[n, b, :len(i1)] = 1.0
            code[n, b, len(i1):c] = 2.0
    return tbl_base, tbl_w, code, cap


_TBL_BASE, _TBL_W, _CODE, _C = _build_tables()
_M = _NB * _C
_CODEF = _CODE.reshape(_N, _M // 128, 128)

_NSPLIT = 2
_NGRP = _N // _NSPLIT
_TASKS_PER_TILE = (_NGRP * _NB) // 32
_BAND_WORDS = _BAND_ROWS * _W


def _sc_sample_body(img_lo, pred_hbm, target_hbm, base_hbm, w00_hbm, w01_hbm,
                    w10_hbm, w11_hbm, out_l_hbm, out_t_hbm,
                    band_p, band_t, base_v, w00_v, w01_v, w10_v, w11_v,
                    out_lv, out_tv, sem):
    wid = lax.axis_index("s") * 2 + lax.axis_index("c")

    def task(t, carry):
        lid = wid * _TASKS_PER_TILE + t
        gid = img_lo * _NB + lid
        n = gid // _NB
        b = gid % _NB
        row0 = jnp.minimum(b * 64, _H - _BAND_ROWS)
        tbl_off = gid * _C
        out_off = lid * _C
        cps = [
            pltpu.async_copy(pred_hbm.at[n, pl.ds(row0, _BAND_ROWS), :],
                             band_p, sem),
            pltpu.async_copy(target_hbm.at[n, pl.ds(row0, _BAND_ROWS), :],
                             band_t, sem),
            pltpu.async_copy(base_hbm.at[pl.ds(tbl_off, _C)], base_v, sem),
            pltpu.async_copy(w00_hbm.at[pl.ds(tbl_off, _C)], w00_v, sem),
            pltpu.async_copy(w01_hbm.at[pl.ds(tbl_off, _C)], w01_v, sem),
            pltpu.async_copy(w10_hbm.at[pl.ds(tbl_off, _C)], w10_v, sem),
            pltpu.async_copy(w11_hbm.at[pl.ds(tbl_off, _C)], w11_v, sem),
        ]
        for cp in cps:
            cp.wait()

        @plsc.parallel_loop(0, _C // 16, unroll=4)
        def group(j):
            sl = pl.ds(j * 16, 16)
            i00 = base_v[sl]
            by = lax.shift_right_logical(i00, 9)
            bx = lax.bitwise_and(i00, 511)
            by1 = by + 1
            bx1 = bx + 1
            a00 = w00_v[sl]
            a01 = w01_v[sl]
            a10 = w10_v[sl]
            a11 = w11_v[sl]
            out_lv[sl] = (plsc.load_gather(band_p, [by, bx]) * a00
                          + plsc.load_gather(band_p, [by, bx1]) * a01
                          + plsc.load_gather(band_p, [by1, bx]) * a10
                          + plsc.load_gather(band_p, [by1, bx1]) * a11)
            out_tv[sl] = (plsc.load_gather(band_t, [by, bx]) * a00
                          + plsc.load_gather(band_t, [by, bx1]) * a01
                          + plsc.load_gather(band_t, [by1, bx]) * a10
                          + plsc.load_gather(band_t, [by1, bx1]) * a11)

        pltpu.sync_copy(out_lv, out_l_hbm.at[pl.ds(out_off, _C)])
        pltpu.sync_copy(out_tv, out_t_hbm.at[pl.ds(out_off, _C)])
        return carry

    lax.fori_loop(0, _TASKS_PER_TILE, task, 0)


@functools.cache
def _sc_sample(img_lo):
    mesh = plsc.VectorSubcoreMesh(core_axis_name="c", subcore_axis_name="s",
                                  num_cores=2, num_subcores=16)
    return pl.kernel(
        functools.partial(_sc_sample_body, img_lo),
        out_type=(jax.ShapeDtypeStruct((_NGRP * _NB * _C,), jnp.float32),
                  jax.ShapeDtypeStruct((_NGRP * _NB * _C,), jnp.float32)),
        mesh=mesh,
        compiler_params=pltpu.CompilerParams(needs_layout_passes=False),
        scratch_types=[
            pltpu.VMEM((_BAND_ROWS, _W), jnp.float32),
            pltpu.VMEM((_BAND_ROWS, _W), jnp.float32),
            pltpu.VMEM((_C,), jnp.int32),
            pltpu.VMEM((_C,), jnp.float32),
            pltpu.VMEM((_C,), jnp.float32),
            pltpu.VMEM((_C,), jnp.float32),
            pltpu.VMEM((_C,), jnp.float32),
            pltpu.VMEM((_C,), jnp.float32),
            pltpu.VMEM((_C,), jnp.float32),
            pltpu.SemaphoreType.DMA,
        ],
    )


_IB = 32


def _tc_reduce(logits_ref, labels_ref, code_ref, bce_ref, dice_ref):
    i = pl.program_id(0)
    l = logits_ref[...]
    t = labels_ref[...]
    codev = code_ref[...]
    cand = codev == 1.0
    alw = codev == 2.0
    absl = jnp.abs(l)
    bits = lax.bitcast_convert_type(absl, jnp.int32)
    bits = jnp.where(cand, bits, jnp.int32(2**31 - 1))

    def body(_, carry):
        lo, hi = carry
        mid = lo + lax.shift_right_logical(hi - lo, 1)
        cnt = jnp.sum((bits <= mid).astype(jnp.int32), axis=(1, 2), keepdims=True)
        ge = cnt >= _K
        return jnp.where(ge, lo, mid + 1), jnp.where(ge, mid, hi)

    lo0 = jnp.zeros((_IB, 1, 1), jnp.int32)
    hi0 = jnp.full((_IB, 1, 1), 2**31 - 1, jnp.int32)
    _, thr = lax.fori_loop(0, 31, body, (lo0, hi0))

    full = jnp.where((cand & (bits <= thr)) | alw, jnp.float32(1.0), jnp.float32(0.0))
    bce = (jnp.maximum(l, 0.0) - l * t + jnp.log1p(jnp.exp(-absl))) * full
    sig = jnp.float32(1.0) / (jnp.float32(1.0) + jnp.exp(-l))
    s1 = jnp.sum(sig * t * full, axis=(1, 2))
    s2 = jnp.sum(sig * full, axis=(1, 2))
    s3 = jnp.sum(t * full, axis=(1, 2))
    dice = jnp.float32(1.0) - (2.0 * s1 + 1.0) / (s2 + s3 + 1.0)

    @pl.when(i == 0)
    def _():
        bce_ref[...] = jnp.zeros_like(bce_ref)
        dice_ref[...] = jnp.zeros_like(dice_ref)

    bce_ref[...] = bce_ref[...] + jnp.sum(bce)
    dice_ref[...] = dice_ref[...] + jnp.sum(dice)


def kernel(pred, target):
    p = pred.reshape(_N, _H, _W)
    t = target.reshape(_N, _H, _W)
    rpi = _M // 128
    tbl = (_TBL_BASE.reshape(-1), _TBL_W[0].reshape(-1), _TBL_W[1].reshape(-1),
           _TBL_W[2].reshape(-1), _TBL_W[3].reshape(-1))
    bce_sum = jnp.float32(0.0)
    dice_sum = jnp.float32(0.0)
    for g in range(_NSPLIT):
        img_lo = g * _NGRP
        out_l, out_t = _sc_sample(img_lo)(p, t, *tbl)
        bs, ds_ = pl.pallas_call(
            _tc_reduce,
            grid=(_NGRP // _IB,),
            in_specs=[
                pl.BlockSpec((_IB, rpi, 128), lambda i: (i, 0, 0)),
                pl.BlockSpec((_IB, rpi, 128), lambda i: (i, 0, 0)),
                pl.BlockSpec((_IB, rpi, 128), lambda i: (i, 0, 0)),
            ],
            out_specs=[
                pl.BlockSpec((1, 1), lambda i: (0, 0)),
                pl.BlockSpec((1, 1), lambda i: (0, 0)),
            ],
            out_shape=[
                jax.ShapeDtypeStruct((1, 1), jnp.float32),
                jax.ShapeDtypeStruct((1, 1), jnp.float32),
            ],
        )(out_l.reshape(_NGRP, rpi, 128), out_t.reshape(_NGRP, rpi, 128),
          _CODEF[img_lo:img_lo + _NGRP])
        bce_sum = bce_sum + bs[0, 0]
        dice_sum = dice_sum + ds_[0, 0]
    loss_bce = bce_sum / jnp.float32(_N * _P)
    loss_dice = dice_sum / jnp.float32(_N)
    loss = loss_bce + loss_dice
    return loss, loss_bce, loss_dice


# speedup vs baseline: 2.7382x; 1.1788x over previous
import functools

import numpy as np
import jax
import jax.numpy as jnp
from jax import lax
from jax.experimental import pallas as pl
from jax.experimental.pallas import tpu as pltpu
from jax.experimental.pallas import tpu_sc as plsc

_N, _H, _W = 64, 512, 512
_S = 112 * 112 * 3
_P = 112 * 112
_K = int(0.75 * _P)
_R = _P - _K
_NB = 16
_BW = _H // _NB
_BAND_ROWS = 40
_ROW0 = np.array([min(b * _BW, _H - _BAND_ROWS) for b in range(_NB)], dtype=np.int32)


def _threefry2x32(k1, k2, x0, x1):
    k1 = np.uint32(k1)
    k2 = np.uint32(k2)
    x0 = x0.astype(np.uint32).copy()
    x1 = x1.astype(np.uint32).copy()
    ks = [k1, k2, np.uint32(k1 ^ k2 ^ np.uint32(0x1BD11BDA))]
    rot = [(13, 15, 26, 6), (17, 29, 16, 24)]
    x0 = x0 + ks[0]
    x1 = x1 + ks[1]
    for g in range(5):
        for r in rot[g % 2]:
            x0 = x0 + x1
            x1 = (x1 << np.uint32(r)) | (x1 >> np.uint32(32 - r))
            x1 = x0 ^ x1
        x0 = x0 + ks[(g + 1) % 3]
        x1 = x1 + ks[(g + 2) % 3] + np.uint32(g + 1)
    return x0, x1


def _np_uniform(rawkey, shape):
    size = int(np.prod(shape))
    b1, b2 = _threefry2x32(rawkey[0], rawkey[1],
                           np.zeros(size, np.uint32),
                           np.arange(size, dtype=np.uint32))
    bits = b1 ^ b2
    fb = (bits >> np.uint32(9)) | np.uint32(0x3F800000)
    return (fb.view(np.float32) - np.float32(1.0)).reshape(shape)


def _build_tables():
    b1, b2 = _threefry2x32(np.uint32(0), np.uint32(42),
                           np.zeros(2, np.uint32), np.arange(2, dtype=np.uint32))
    pc = _np_uniform((b1[0], b2[0]), (_N, _S, 2))
    rc = _np_uniform((b1[1], b2[1]), (_N, _R, 2))
    coords = np.concatenate([pc, rc], axis=1)

    x = coords[..., 0] * np.float32(_W) - np.float32(0.5)
    y = coords[..., 1] * np.float32(_H) - np.float32(0.5)
    x0 = np.floor(x)
    y0 = np.floor(y)
    fx1 = x - x0
    fx0 = np.float32(1.0) - fx1
    fy1 = y - y0
    fy0 = np.float32(1.0) - fy1

    def slot_weights(c0, f0, f1, lim):
        base = np.clip(c0, 0.0, lim - 2.0).astype(np.int32)
        g = np.zeros(c0.shape + (2,), dtype=np.float32)
        for d, f in ((0, f0), (1, f1)):
            ic = c0 + d
            valid = (ic >= 0) & (ic <= lim - 1)
            slot = np.clip(ic.astype(np.int64) - base, 0, 1).astype(np.int32)
            for s_ in (0, 1):
                g[..., s_] += np.where(valid & (slot == s_), f, np.float32(0.0))
        return base, g

    basex, gx = slot_weights(x0, fx0, fx1, _W)
    basey, gy = slot_weights(y0, fy0, fy1, _H)

    band = np.minimum(basey // _BW, _NB - 1).astype(np.int32)
    base_local = (basey - _ROW0[band]) * _W + basex
    ws = (gx[..., 0] * gy[..., 0], gx[..., 1] * gy[..., 0],
          gx[..., 0] * gy[..., 1], gx[..., 1] * gy[..., 1])

    counts = np.zeros((_N, _NB), np.int32)
    for n in range(_N):
        counts[n] = np.bincount(band[n], minlength=_NB)
    cap = int(counts.max())
    cap = (cap + 127) // 128 * 128

    tbl_base = np.zeros((_N, _NB, cap), np.int32)
    tbl_w = np.zeros((4, _N, _NB, cap), np.float32)
    code = np.zeros((_N, _NB, cap), np.float32)
    for n in range(_N):
        for b in range(_NB):
            i1 = np.nonzero(band[n, :_S] == b)[0]
            i2 = np.nonzero(band[n, _S:] == b)[0] + _S
            idx = np.concatenate([i1, i2])
            c = len(idx)
            tbl_base[n, b, :c] = base_local[n, idx]
            for k_ in range(4):
                tbl_w[k_, n, b, :c] = ws[k_][n, idx]
            code[n, b, :len(i1)] = 1.0
            code[n, b, len(i1):c] = 2.0
    return tbl_base, tbl_w, code, cap


_TBL_BASE, _TBL_W, _CODE, _C = _build_tables()
_M = _NB * _C
_CODEF = _CODE.reshape(_N, _M // 128, 128)

_NSPLIT = 2
_NGRP = _N // _NSPLIT
_TASKS_PER_TILE = (_NGRP * _NB) // 32
_BAND_WORDS = _BAND_ROWS * _W


def _sc_sample_body(img_lo, pred_hbm, target_hbm, base_hbm, w00_hbm, w01_hbm,
                    w10_hbm, w11_hbm, out_l_hbm, out_t_hbm, *scr):
    s0, s1 = scr[0:9], scr[9:18]
    isem0, isem1, osem0, osem1 = scr[18:22]
    wid = lax.axis_index("s") * 2 + lax.axis_index("c")
    T = _TASKS_PER_TILE

    def build_in(t, S, sem):
        band_p, band_t, base_v, w00_v, w01_v, w10_v, w11_v = S[:7]
        lid = wid * T + t
        gid = img_lo * _NB + lid
        n = gid // _NB
        b = gid % _NB
        row0 = jnp.minimum(b * _BW, _H - _BAND_ROWS)
        tbl_off = gid * _C
        mk = pltpu.make_async_copy
        return [
            mk(pred_hbm.at[n, pl.ds(row0, _BAND_ROWS), :], band_p, sem),
            mk(target_hbm.at[n, pl.ds(row0, _BAND_ROWS), :], band_t, sem),
            mk(base_hbm.at[pl.ds(tbl_off, _C)], base_v, sem),
            mk(w00_hbm.at[pl.ds(tbl_off, _C)], w00_v, sem),
            mk(w01_hbm.at[pl.ds(tbl_off, _C)], w01_v, sem),
            mk(w10_hbm.at[pl.ds(tbl_off, _C)], w10_v, sem),
            mk(w11_hbm.at[pl.ds(tbl_off, _C)], w11_v, sem),
        ]

    def build_out(t, S, sem):
        out_lv, out_tv = S[7], S[8]
        out_off = (wid * T + t) * _C
        mk = pltpu.make_async_copy
        return [
            mk(out_lv, out_l_hbm.at[pl.ds(out_off, _C)], sem),
            mk(out_tv, out_t_hbm.at[pl.ds(out_off, _C)], sem),
        ]

    def compute(S):
        band_p, band_t, base_v, w00_v, w01_v, w10_v, w11_v, out_lv, out_tv = S

        @plsc.parallel_loop(0, _C // 16, unroll=4)
        def group(j):
            sl = pl.ds(j * 16, 16)
            i00 = base_v[sl]
            by = lax.shift_right_logical(i00, 9)
            bx = lax.bitwise_and(i00, 511)
            by1 = by + 1
            bx1 = bx + 1
            a00 = w00_v[sl]
            a01 = w01_v[sl]
            a10 = w10_v[sl]
            a11 = w11_v[sl]
            out_lv[sl] = (plsc.load_gather(band_p, [by, bx]) * a00
                          + plsc.load_gather(band_p, [by, bx1]) * a01
                          + plsc.load_gather(band_p, [by1, bx]) * a10
                          + plsc.load_gather(band_p, [by1, bx1]) * a11)
            out_tv[sl] = (plsc.load_gather(band_t, [by, bx]) * a00
                          + plsc.load_gather(band_t, [by, bx1]) * a01
                          + plsc.load_gather(band_t, [by1, bx]) * a10
                          + plsc.load_gather(band_t, [by1, bx1]) * a11)

    for cp in build_in(0, s0, isem0):
        cp.start()

    def pair(i2, carry):
        t0 = i2 * 2
        for cp in build_in(t0, s0, isem0):
            cp.wait()
        for cp in build_in(t0 + 1, s1, isem1):
            cp.start()

        @pl.when(i2 > 0)
        def _():
            for cp in build_out(t0, s0, osem0):
                cp.wait()

        compute(s0)
        for cp in build_out(t0, s0, osem0):
            cp.start()

        for cp in build_in(t0 + 1, s1, isem1):
            cp.wait()

        @pl.when(t0 + 2 < T)
        def _():
            for cp in build_in(t0 + 2, s0, isem0):
                cp.start()

        @pl.when(i2 > 0)
        def _():
            for cp in build_out(t0 + 1, s1, osem1):
                cp.wait()

        compute(s1)
        for cp in build_out(t0 + 1, s1, osem1):
            cp.start()
        return carry

    lax.fori_loop(0, T // 2, pair, 0)
    for cp in build_out(T - 2, s0, osem0):
        cp.wait()
    for cp in build_out(T - 1, s1, osem1):
        cp.wait()


@functools.cache
def _sc_sample(img_lo):
    mesh = plsc.VectorSubcoreMesh(core_axis_name="c", subcore_axis_name="s",
                                  num_cores=2, num_subcores=16)
    bufset = [
        pltpu.VMEM((_BAND_ROWS, _W), jnp.float32),
        pltpu.VMEM((_BAND_ROWS, _W), jnp.float32),
        pltpu.VMEM((_C,), jnp.int32),
        pltpu.VMEM((_C,), jnp.float32),
        pltpu.VMEM((_C,), jnp.float32),
        pltpu.VMEM((_C,), jnp.float32),
        pltpu.VMEM((_C,), jnp.float32),
        pltpu.VMEM((_C,), jnp.float32),
        pltpu.VMEM((_C,), jnp.float32),
    ]
    return pl.kernel(
        functools.partial(_sc_sample_body, img_lo),
        out_type=(jax.ShapeDtypeStruct((_NGRP * _NB * _C,), jnp.float32),
                  jax.ShapeDtypeStruct((_NGRP * _NB * _C,), jnp.float32)),
        mesh=mesh,
        compiler_params=pltpu.CompilerParams(needs_layout_passes=False),
        scratch_types=bufset + bufset + [pltpu.SemaphoreType.DMA] * 4,
    )


_IB = 32


def _tc_reduce(logits_ref, labels_ref, code_ref, bce_ref, dice_ref):
    i = pl.program_id(0)
    l = logits_ref[...]
    t = labels_ref[...]
    codev = code_ref[...]
    cand = codev == 1.0
    alw = codev == 2.0
    absl = jnp.abs(l)
    bits = lax.bitcast_convert_type(absl, jnp.int32)
    bits = jnp.where(cand, bits, jnp.int32(2**31 - 1))

    def body(_, carry):
        lo, hi = carry
        mid = lo + lax.shift_right_logical(hi - lo, 1)
        cnt = jnp.sum((bits <= mid).astype(jnp.int32), axis=(1, 2), keepdims=True)
        ge = cnt >= _K
        return jnp.where(ge, lo, mid + 1), jnp.where(ge, mid, hi)

    lo0 = jnp.zeros((_IB, 1, 1), jnp.int32)
    hi0 = jnp.full((_IB, 1, 1), 2**31 - 1, jnp.int32)
    _, thr = lax.fori_loop(0, 31, body, (lo0, hi0))

    full = jnp.where((cand & (bits <= thr)) | alw, jnp.float32(1.0), jnp.float32(0.0))
    bce = (jnp.maximum(l, 0.0) - l * t + jnp.log1p(jnp.exp(-absl))) * full
    sig = jnp.float32(1.0) / (jnp.float32(1.0) + jnp.exp(-l))
    s1 = jnp.sum(sig * t * full, axis=(1, 2))
    s2 = jnp.sum(sig * full, axis=(1, 2))
    s3 = jnp.sum(t * full, axis=(1, 2))
    dice = jnp.float32(1.0) - (2.0 * s1 + 1.0) / (s2 + s3 + 1.0)

    @pl.when(i == 0)
    def _():
        bce_ref[...] = jnp.zeros_like(bce_ref)
        dice_ref[...] = jnp.zeros_like(dice_ref)

    bce_ref[...] = bce_ref[...] + jnp.sum(bce)
    dice_ref[...] = dice_ref[...] + jnp.sum(dice)


def kernel(pred, target):
    p = pred.reshape(_N, _H, _W)
    t = target.reshape(_N, _H, _W)
    rpi = _M // 128
    tbl = (_TBL_BASE.reshape(-1), _TBL_W[0].reshape(-1), _TBL_W[1].reshape(-1),
           _TBL_W[2].reshape(-1), _TBL_W[3].reshape(-1))
    bce_sum = jnp.float32(0.0)
    dice_sum = jnp.float32(0.0)
    for g in range(_NSPLIT):
        img_lo = g * _NGRP
        out_l, out_t = _sc_sample(img_lo)(p, t, *tbl)
        bs, ds_ = pl.pallas_call(
            _tc_reduce,
            grid=(_NGRP // _IB,),
            in_specs=[
                pl.BlockSpec((_IB, rpi, 128), lambda i: (i, 0, 0)),
                pl.BlockSpec((_IB, rpi, 128), lambda i: (i, 0, 0)),
                pl.BlockSpec((_IB, rpi, 128), lambda i: (i, 0, 0)),
            ],
            out_specs=[
                pl.BlockSpec((1, 1), lambda i: (0, 0)),
                pl.BlockSpec((1, 1), lambda i: (0, 0)),
            ],
            out_shape=[
                jax.ShapeDtypeStruct((1, 1), jnp.float32),
                jax.ShapeDtypeStruct((1, 1), jnp.float32),
            ],
        )(out_l.reshape(_NGRP, rpi, 128), out_t.reshape(_NGRP, rpi, 128),
          _CODEF[img_lo:img_lo + _NGRP])
        bce_sum = bce_sum + bs[0, 0]
        dice_sum = dice_sum + ds_[0, 0]
    loss_bce = bce_sum / jnp.float32(_N * _P)
    loss_dice = dice_sum / jnp.float32(_N)
    loss = loss_bce + loss_dice
    return loss, loss_bce, loss_dice
